# Initial kernel scaffold; baseline (speedup 1.0000x reference)
import jax
import jax.numpy as jnp
from jax.experimental import pallas as pl


def kernel(x, edge_index, edge_weights, feature_mask, W, b):
    raise NotImplementedError("write your pallas kernel here")


# speedup vs baseline: 1.0768x; 1.0768x over previous
import functools

import jax
import jax.numpy as jnp
from jax import lax
from jax.experimental import pallas as pl
from jax.experimental.pallas import tpu as pltpu
from jax.experimental.pallas import tpu_sc as plsc

N = 10000
E = 320000
D = 128
L = 3
K = 3
NEG_SLOPE = 0.01

NC = 2
NS = 16
NW = NC * NS
CHUNK = 128
EPW = ((E + NW * CHUNK - 1) // (NW * CHUNK)) * CHUNK
EPAD = EPW * NW
NPAD16 = ((N + 15) // 16) * 16
ROWS_PER_SUB = 624
ROW_BLK = 2000

_mesh = plsc.VectorSubcoreMesh(core_axis_name="c", subcore_axis_name="s")
_sc_params = pltpu.CompilerParams(needs_layout_passes=False)


@functools.partial(
    pl.kernel,
    mesh=_mesh,
    out_type=jax.ShapeDtypeStruct((NW * NPAD16,), jnp.float32),
    scratch_types=[
        pltpu.VMEM((NPAD16,), jnp.float32),
        pltpu.VMEM((EPW,), jnp.int32),
        pltpu.VMEM((EPW,), jnp.float32),
    ],
    compiler_params=_sc_params,
)
def _deg_kernel(col_hbm, w_hbm, out_hbm, deg_v, col_v, w_v):
    cid = lax.axis_index("c")
    sid = lax.axis_index("s")
    ew = cid * NS + sid
    zeros = jnp.zeros((16,), jnp.float32)

    def zb(i, carry):
        deg_v[pl.ds(i * 16, 16)] = zeros
        return carry

    lax.fori_loop(0, NPAD16 // 16, zb, None)
    pltpu.sync_copy(col_hbm.at[pl.ds(ew * EPW, EPW)], col_v)
    pltpu.sync_copy(w_hbm.at[pl.ds(ew * EPW, EPW)], w_v)

    def body(g, carry):
        cvec = col_v[pl.ds(g * 16, 16)]
        wvec = w_v[pl.ds(g * 16, 16)]
        plsc.addupdate_scatter(deg_v, [cvec], wvec)
        return carry

    lax.fori_loop(0, EPW // 16, body, None)
    pltpu.sync_copy(deg_v, out_hbm.at[pl.ds(ew * NPAD16, NPAD16)])


@functools.partial(
    pl.kernel,
    mesh=_mesh,
    out_type=jax.ShapeDtypeStruct((NC * N, D), jnp.float32),
    scratch_types=[
        pltpu.VMEM_SHARED((N, D), jnp.float32),
        pltpu.VMEM((CHUNK, D), jnp.float32),
        pltpu.VMEM((CHUNK,), jnp.int32),
        pltpu.VMEM((CHUNK,), jnp.int32),
        pltpu.VMEM((CHUNK,), jnp.float32),
        pltpu.SemaphoreType.DMA,
    ],
    compiler_params=_sc_params,
)
def _prop_kernel(u_hbm, row_hbm, col_hbm, w_hbm, z_hbm, out_hbm,
                 acc_sh, rows_v, ridx_v, cidx_v, w_v, sem):
    cid = lax.axis_index("c")
    sid = lax.axis_index("s")
    ew = cid * NS + sid
    pltpu.sync_copy(z_hbm, acc_sh.at[pl.ds(sid * ROWS_PER_SUB, ROWS_PER_SUB)])

    @pl.when(sid == NS - 1)
    def _zero_tail():
        pltpu.sync_copy(z_hbm.at[pl.ds(0, 16)],
                        acc_sh.at[pl.ds(NS * ROWS_PER_SUB, 16)])

    plsc.subcore_barrier()
    lanes = lax.iota(jnp.int32, 16)

    def chunk_body(j, carry):
        off = ew * EPW + j * CHUNK
        pltpu.sync_copy(row_hbm.at[pl.ds(off, CHUNK)], ridx_v)
        pltpu.sync_copy(col_hbm.at[pl.ds(off, CHUNK)], cidx_v)
        pltpu.sync_copy(w_hbm.at[pl.ds(off, CHUNK)], w_v)
        pltpu.async_copy(u_hbm.at[ridx_v], rows_v, sem).wait()
        for c0 in range(0, CHUNK, 16):
            wv = w_v[pl.ds(c0, 16)]
            cvec = lanes + c0

            def fbody(f, carry2):
                fvec = jnp.full((16,), f, jnp.int32)
                vals = plsc.load_gather(rows_v, [cvec, fvec])
                plsc.store_scatter(rows_v, [cvec, fvec], vals * wv)
                return carry2

            lax.fori_loop(0, D, fbody, None, unroll=16)
        pltpu.sync_copy(rows_v, acc_sh.at[cidx_v], add=True)
        return carry

    lax.fori_loop(0, EPW // CHUNK, chunk_body, None)
    plsc.subcore_barrier()
    pltpu.sync_copy(
        acc_sh.at[pl.ds(sid * ROWS_PER_SUB, ROWS_PER_SUB)],
        out_hbm.at[pl.ds(cid * N + sid * ROWS_PER_SUB, ROWS_PER_SUB)],
    )

    @pl.when(sid == NS - 1)
    def _out_tail():
        pltpu.sync_copy(
            acc_sh.at[pl.ds(NS * ROWS_PER_SUB, 16)],
            out_hbm.at[pl.ds(cid * N + NS * ROWS_PER_SUB, 16)],
        )


def _prep_body(parts_ref, x_ref, dis_ref, u0_ref):
    deg = jnp.sum(parts_ref[...], axis=0)
    dis = jnp.where(deg > 0, lax.rsqrt(jnp.where(deg > 0, deg, 1.0)), 0.0)
    dis_ref[...] = dis[:, None]
    u0_ref[...] = x_ref[...] * dis[:, None]


_prep = pl.pallas_call(
    _prep_body,
    out_shape=[
        jax.ShapeDtypeStruct((N, 1), jnp.float32),
        jax.ShapeDtypeStruct((N, D), jnp.float32),
    ],
)


def _comb_body(p_ref, dis_ref, h_ref, u_ref):
    s = p_ref[0] + p_ref[1]
    dis = dis_ref[...]
    h = s * dis
    h_ref[...] = h
    u_ref[...] = h * dis


_dspec = pl.BlockSpec((ROW_BLK, 1), lambda i: (i, 0))
_comb = pl.pallas_call(
    _comb_body,
    grid=(N // ROW_BLK,),
    in_specs=[
        pl.BlockSpec((NC, ROW_BLK, D), lambda i: (0, i, 0)),
        _dspec,
    ],
    out_specs=[
        pl.BlockSpec((ROW_BLK, D), lambda i: (i, 0)),
        pl.BlockSpec((ROW_BLK, D), lambda i: (i, 0)),
    ],
    out_shape=[
        jax.ShapeDtypeStruct((N, D), jnp.float32),
        jax.ShapeDtypeStruct((N, D), jnp.float32),
    ],
)


def _acc4(h_refs, w_ref, b_ref):
    acc = b_ref[...][None, :].astype(jnp.float32)
    for k in range(K + 1):
        acc = acc + jnp.dot(h_refs[k][...], w_ref[k],
                            preferred_element_type=jnp.float32)
    return acc


def _mid_body(h0, h1, h2, h3, w_ref, b_ref, dis_ref, h_ref, u_ref):
    acc = _acc4((h0, h1, h2, h3), w_ref, b_ref)
    h = jnp.where(acc > 0, acc, NEG_SLOPE * acc)
    h_ref[...] = h
    u_ref[...] = h * dis_ref[...]


_hspec = pl.BlockSpec((ROW_BLK, D), lambda i: (i, 0))
_mid = pl.pallas_call(
    _mid_body,
    grid=(N // ROW_BLK,),
    in_specs=[_hspec, _hspec, _hspec, _hspec,
              pl.BlockSpec((K + 1, D, D), lambda i: (0, 0, 0)),
              pl.BlockSpec((D,), lambda i: (0,)),
              _dspec],
    out_specs=[_hspec, _hspec],
    out_shape=[
        jax.ShapeDtypeStruct((N, D), jnp.float32),
        jax.ShapeDtypeStruct((N, D), jnp.float32),
    ],
)


def _last_body(h0, h1, h2, h3, w_ref, b_ref, mask_ref, o_ref):
    acc = _acc4((h0, h1, h2, h3), w_ref, b_ref)
    o_ref[...] = acc * mask_ref[...]


_last = pl.pallas_call(
    _last_body,
    grid=(N // ROW_BLK,),
    in_specs=[_hspec, _hspec, _hspec, _hspec,
              pl.BlockSpec((K + 1, D, D), lambda i: (0, 0, 0)),
              pl.BlockSpec((D,), lambda i: (0,)),
              _dspec],
    out_specs=_hspec,
    out_shape=jax.ShapeDtypeStruct((N, D), jnp.float32),
)


def kernel(x, edge_index, edge_weights, feature_mask, W, b):
    pad = EPAD - E
    row_p = jnp.concatenate([edge_index[0], jnp.zeros((pad,), jnp.int32)])
    col_p = jnp.concatenate([edge_index[1], jnp.zeros((pad,), jnp.int32)])
    w_p = jnp.concatenate([edge_weights, jnp.zeros((pad,), jnp.float32)])
    z625 = jnp.zeros((ROWS_PER_SUB, D), jnp.float32)

    parts_deg = _deg_kernel(col_p, w_p).reshape(NW, NPAD16)[:, :N]
    dis, u = _prep(parts_deg, x)

    h = x
    out = None
    for l in range(L):
        hs = [h]
        ucur = u
        for _ in range(K):
            part = _prop_kernel(ucur, row_p, col_p, w_p, z625)
            hk, ucur = _comb(part.reshape(NC, N, D), dis)
            hs.append(hk)
        if l < L - 1:
            h, u = _mid(hs[0], hs[1], hs[2], hs[3], W[l], b[l], dis)
        else:
            out = _last(hs[0], hs[1], hs[2], hs[3], W[l], b[l],
                        feature_mask[:, None])
    return out
